# Initial kernel scaffold; baseline (speedup 1.0000x reference)
import jax
import jax.numpy as jnp
from jax.experimental import pallas as pl


def kernel(x, params):
    raise NotImplementedError("write your pallas kernel here")


# speedup vs baseline: 1.0180x; 1.0180x over previous
import math
from functools import partial

import jax
import jax.numpy as jnp
from jax import lax
from jax.experimental import pallas as pl

EPS_BN = 1e-5
EPS_LN = 1e-5


def _conv(x, w, s, p):
    return lax.conv_general_dilated(
        x, w, (s, s), [(p, p), (p, p)], dimension_numbers=('NCHW', 'OIHW', 'NCHW'))


def _bn(x, p):
    return (x / jnp.sqrt(1.0 + EPS_BN)) * p['g'].reshape(1, -1, 1, 1) + p['b'].reshape(1, -1, 1, 1)


def _se(x, p):
    y = x.mean((2, 3))
    y = jax.nn.relu(y @ p['se1'].T)
    y = jax.nn.sigmoid(y @ p['se2'].T)
    return x * y[:, :, None, None]


def _block(x, p, stride):
    out = jax.nn.relu(_bn(_conv(x, p['conv1'], stride, 1), p['bn1']))
    out = _bn(_conv(out, p['conv2'], 1, 1), p['bn2'])
    out = _se(out, p)
    sc = x if 'sc_conv' not in p else _bn(_conv(x, p['sc_conv'], stride, 0), p['sc_bn'])
    return jax.nn.relu(out + sc)


def _maxpool(x):
    return lax.reduce_window(x, -jnp.inf, lax.max, (1, 1, 3, 3), (1, 1, 2, 2),
                             [(0, 0), (0, 0), (1, 1), (1, 1)])


def _features(x, p):
    x = jax.nn.relu(_bn(_conv(x, p['conv1'], 2, 3), p['bn1']))
    x = _maxpool(x)
    for name, s in [('layer1', 1), ('layer2', 2), ('layer3', 2), ('layer4', 2)]:
        x = _block(x, p[name][0], s)
        x = _block(x, p[name][1], 1)
    return x.mean((2, 3))


_SQRT2 = math.sqrt(2.0)
_NEXP = 8


def _dot_t(a, w):
    return lax.dot_general(a, w, (((1,), (1,)), ((), ())),
                           preferred_element_type=jnp.float32)


def _ln_k(x, g, b):
    m = x.mean(-1, keepdims=True)
    v = ((x - m) ** 2).mean(-1, keepdims=True)
    return (x - m) / jnp.sqrt(v + EPS_LN) * g + b


def _gelu_k(x):
    return 0.5 * x * (1.0 + lax.erf(x / _SQRT2))


def _moe_head_kernel(feats_ref, *refs):
    out_ref = refs[-1]
    grefs = refs[:11]
    erefs = refs[11:-1]

    feats = feats_ref[...]

    w1, b1, g1, bb1, w2, b2, g2, bb2, w3, b3, temp = (r[...] for r in grefs)
    h = _gelu_k(_ln_k(_dot_t(feats, w1) + b1, g1, bb1))
    h = _gelu_k(_ln_k(_dot_t(h, w2) + b2, g2, bb2))
    logits = _dot_t(h, w3) + b3
    logits = logits / jnp.maximum(temp[0, 0], 0.1)

    cols = lax.broadcasted_iota(jnp.int32, logits.shape, 1)
    m1 = jnp.max(logits, axis=-1, keepdims=True)
    i1 = jnp.min(jnp.where(logits == m1, cols, _NEXP), axis=-1, keepdims=True)
    l2 = jnp.where(cols == i1, -1e30, logits)
    m2 = jnp.max(l2, axis=-1, keepdims=True)
    i2 = jnp.min(jnp.where(l2 == m2, cols, _NEXP), axis=-1, keepdims=True)
    e2 = jnp.exp(m2 - m1)
    denom = 1.0 + e2
    g0 = 1.0 / denom
    g1v = e2 / denom
    s = g0 + g1v + 1e-8
    g0 = g0 / s
    g1v = g1v / s
    gates = jnp.where(cols == i1, g0, 0.0) + jnp.where(cols == i2, g1v, 0.0)

    acc = jnp.zeros(out_ref.shape, jnp.float32)
    for e in range(_NEXP):
        (ew1, eb1, eg1, ebb1, ew2, eb2, eg2, ebb2,
         ew3, eb3, eg3, ebb3, ew4, eb4) = (r[...] for r in erefs[14 * e:14 * (e + 1)])
        hh = _gelu_k(_ln_k(_dot_t(feats, ew1) + eb1, eg1, ebb1))
        hh = _gelu_k(_ln_k(_dot_t(hh, ew2) + eb2, eg2, ebb2))
        hh = _gelu_k(_ln_k(_dot_t(hh, ew3) + eb3, eg3, ebb3))
        y = _dot_t(hh, ew4) + eb4
        acc = acc + gates[:, e:e + 1] * y
    out_ref[...] = acc


def _row(v):
    return v.reshape(1, -1)


def _moe_head(feats, gate, experts):
    B = feats.shape[0]
    gargs = [gate['fc1']['w'], _row(gate['fc1']['b']), _row(gate['ln1']['g']), _row(gate['ln1']['b']),
             gate['fc2']['w'], _row(gate['fc2']['b']), _row(gate['ln2']['g']), _row(gate['ln2']['b']),
             gate['fc3']['w'], _row(gate['fc3']['b']), _row(gate['temp'])]
    eargs = []
    for ep in experts:
        eargs += [ep['fc1']['w'], _row(ep['fc1']['b']), _row(ep['ln1']['g']), _row(ep['ln1']['b']),
                  ep['fc2']['w'], _row(ep['fc2']['b']), _row(ep['ln2']['g']), _row(ep['ln2']['b']),
                  ep['fc3']['w'], _row(ep['fc3']['b']), _row(ep['ln3']['g']), _row(ep['ln3']['b']),
                  ep['fc4']['w'], _row(ep['fc4']['b'])]
    nout = experts[0]['fc4']['w'].shape[0]
    return pl.pallas_call(
        _moe_head_kernel,
        out_shape=jax.ShapeDtypeStruct((B, nout), jnp.float32),
    )(feats, *gargs, *eargs)


def kernel(x, params):
    feats = _features(x, params['fe'])
    return _moe_head(feats, params['gate'], params['experts'])


# speedup vs baseline: 1.0214x; 1.0034x over previous
import math
from functools import partial

import jax
import jax.numpy as jnp
from jax import lax
from jax.experimental import pallas as pl

EPS_BN = 1e-5
EPS_LN = 1e-5


_BF = jnp.bfloat16


def _conv_bn(x, w, bn, s, p):
    scale = bn['g'] * (1.0 / math.sqrt(1.0 + EPS_BN))
    wf = (w * scale[:, None, None, None]).astype(_BF)
    y = lax.conv_general_dilated(
        x, wf, (s, s), [(p, p), (p, p)], dimension_numbers=('NCHW', 'OIHW', 'NCHW'))
    return y + bn['b'].astype(_BF).reshape(1, -1, 1, 1)


def _se(x, p):
    y = x.astype(jnp.float32).mean((2, 3))
    y = jax.nn.relu(y @ p['se1'].T)
    y = jax.nn.sigmoid(y @ p['se2'].T)
    return x * y.astype(_BF)[:, :, None, None]


def _block(x, p, stride):
    out = jax.nn.relu(_conv_bn(x, p['conv1'], p['bn1'], stride, 1))
    out = _conv_bn(out, p['conv2'], p['bn2'], 1, 1)
    out = _se(out, p)
    sc = x if 'sc_conv' not in p else _conv_bn(x, p['sc_conv'], p['sc_bn'], stride, 0)
    return jax.nn.relu(out + sc)


def _maxpool(x):
    return lax.reduce_window(x, _BF(-jnp.inf), lax.max, (1, 1, 3, 3), (1, 1, 2, 2),
                             [(0, 0), (0, 0), (1, 1), (1, 1)])


def _features(x, p):
    x = x.astype(_BF)
    x = jax.nn.relu(_conv_bn(x, p['conv1'], p['bn1'], 2, 3))
    x = _maxpool(x)
    for name, s in [('layer1', 1), ('layer2', 2), ('layer3', 2), ('layer4', 2)]:
        x = _block(x, p[name][0], s)
        x = _block(x, p[name][1], 1)
    return x.astype(jnp.float32).mean((2, 3))


_SQRT2 = math.sqrt(2.0)
_NEXP = 8


def _dot_t(a, w):
    return lax.dot_general(a, w, (((1,), (1,)), ((), ())),
                           preferred_element_type=jnp.float32)


def _ln_k(x, g, b):
    m = x.mean(-1, keepdims=True)
    v = ((x - m) ** 2).mean(-1, keepdims=True)
    return (x - m) / jnp.sqrt(v + EPS_LN) * g + b


def _gelu_k(x):
    return 0.5 * x * (1.0 + lax.erf(x / _SQRT2))


def _moe_head_kernel(feats_ref, *refs):
    out_ref = refs[-1]
    grefs = refs[:11]
    erefs = refs[11:-1]

    feats = feats_ref[...]

    w1, b1, g1, bb1, w2, b2, g2, bb2, w3, b3, temp = (r[...] for r in grefs)
    h = _gelu_k(_ln_k(_dot_t(feats, w1) + b1, g1, bb1))
    h = _gelu_k(_ln_k(_dot_t(h, w2) + b2, g2, bb2))
    logits = _dot_t(h, w3) + b3
    logits = logits / jnp.maximum(temp[0, 0], 0.1)

    cols = lax.broadcasted_iota(jnp.int32, logits.shape, 1)
    m1 = jnp.max(logits, axis=-1, keepdims=True)
    i1 = jnp.min(jnp.where(logits == m1, cols, _NEXP), axis=-1, keepdims=True)
    l2 = jnp.where(cols == i1, -1e30, logits)
    m2 = jnp.max(l2, axis=-1, keepdims=True)
    i2 = jnp.min(jnp.where(l2 == m2, cols, _NEXP), axis=-1, keepdims=True)
    e2 = jnp.exp(m2 - m1)
    denom = 1.0 + e2
    g0 = 1.0 / denom
    g1v = e2 / denom
    s = g0 + g1v + 1e-8
    g0 = g0 / s
    g1v = g1v / s
    gates = jnp.where(cols == i1, g0, 0.0) + jnp.where(cols == i2, g1v, 0.0)

    acc = jnp.zeros(out_ref.shape, jnp.float32)
    for e in range(_NEXP):
        (ew1, eb1, eg1, ebb1, ew2, eb2, eg2, ebb2,
         ew3, eb3, eg3, ebb3, ew4, eb4) = (r[...] for r in erefs[14 * e:14 * (e + 1)])
        hh = _gelu_k(_ln_k(_dot_t(feats, ew1) + eb1, eg1, ebb1))
        hh = _gelu_k(_ln_k(_dot_t(hh, ew2) + eb2, eg2, ebb2))
        hh = _gelu_k(_ln_k(_dot_t(hh, ew3) + eb3, eg3, ebb3))
        y = _dot_t(hh, ew4) + eb4
        acc = acc + gates[:, e:e + 1] * y
    out_ref[...] = acc


def _row(v):
    return v.reshape(1, -1)


def _moe_head(feats, gate, experts):
    B = feats.shape[0]
    gargs = [gate['fc1']['w'], _row(gate['fc1']['b']), _row(gate['ln1']['g']), _row(gate['ln1']['b']),
             gate['fc2']['w'], _row(gate['fc2']['b']), _row(gate['ln2']['g']), _row(gate['ln2']['b']),
             gate['fc3']['w'], _row(gate['fc3']['b']), _row(gate['temp'])]
    eargs = []
    for ep in experts:
        eargs += [ep['fc1']['w'], _row(ep['fc1']['b']), _row(ep['ln1']['g']), _row(ep['ln1']['b']),
                  ep['fc2']['w'], _row(ep['fc2']['b']), _row(ep['ln2']['g']), _row(ep['ln2']['b']),
                  ep['fc3']['w'], _row(ep['fc3']['b']), _row(ep['ln3']['g']), _row(ep['ln3']['b']),
                  ep['fc4']['w'], _row(ep['fc4']['b'])]
    nout = experts[0]['fc4']['w'].shape[0]
    return pl.pallas_call(
        _moe_head_kernel,
        out_shape=jax.ShapeDtypeStruct((B, nout), jnp.float32),
    )(feats, *gargs, *eargs)


def kernel(x, params):
    feats = _features(x, params['fe'])
    return _moe_head(feats, params['gate'], params['experts'])
